# Initial kernel scaffold; baseline (speedup 1.0000x reference)
import jax
import jax.numpy as jnp
from jax.experimental import pallas as pl


def kernel(input_ids, weights):
    raise NotImplementedError("write your pallas kernel here")


# speedup vs baseline: 1.6292x; 1.6292x over previous
import functools

import jax
import jax.numpy as jnp
from jax import lax
from jax.experimental import pallas as pl
from jax.experimental.pallas import tpu as pltpu
from jax.experimental.pallas import tpu_sc as plsc

_PAD = 1

_B, _S, _D = 4, 4096, 1024
_NC, _NS = 2, 16
_NW = _NC * _NS
_CH = (_B * _S) // _NW
_ROWS_PER_CORE = _B // _NC
_CHUNKS_PER_ROW = _NS // _ROWS_PER_CORE
_G = 32
_NG = _CH // _G
_L = 16

_LANE = None


def _lane_cumsum(x):
    lane = lax.iota(jnp.int32, _L)
    cs = x
    for d in (1, 2, 4, 8):
        idx = jnp.maximum(lane - d, 0)
        sh = cs.at[idx].get(mode="promise_in_bounds")
        cs = cs + jnp.where(lane >= d, sh, 0)
    return cs


def _splat_last(x):
    return x.at[jnp.full((_L,), _L - 1, jnp.int32)].get(
        mode="promise_in_bounds")


def _sc_body(ids_hbm, w_hbm, out_hbm, ids_v, idx_v, rows0_v, rows1_v,
             sem0, sem1):
    c = lax.axis_index("c")
    s = lax.axis_index("s")
    row = c * _ROWS_PER_CORE + s // _CHUNKS_PER_ROW
    chunk = s % _CHUNKS_PER_ROW

    pltpu.sync_copy(ids_hbm.at[row], ids_v)

    def pref_body(j, acc):
        v = ids_v[pl.ds(j * _L, _L)]
        return acc + jnp.where(v != _PAD, jnp.int32(1), jnp.int32(0))

    acc = lax.fori_loop(0, chunk * (_CH // _L), pref_body,
                        jnp.zeros((_L,), jnp.int32))
    offset = _splat_last(_lane_cumsum(acc))

    chunk_base = chunk * _CH

    def pos_body(j, off):
        v = ids_v[pl.ds(chunk_base + j * _L, _L)]
        mi = jnp.where(v != _PAD, jnp.int32(1), jnp.int32(0))
        cs = _lane_cumsum(mi) + off
        idx_v[pl.ds(j * _L, _L)] = cs * mi + _PAD
        return _splat_last(cs)

    lax.fori_loop(0, _CH // _L, pos_body, offset)

    out_base = row * _S + chunk_base
    bufs = (rows0_v, rows1_v)
    sems = (sem0, sem1)
    descs = [None, None]
    descs[0] = pltpu.async_copy(w_hbm.at[idx_v.at[pl.ds(0, _G)]],
                                bufs[0], sems[0])
    for g in range(_NG):
        b = g % 2
        if g + 1 < _NG:
            descs[1 - b] = pltpu.async_copy(
                w_hbm.at[idx_v.at[pl.ds((g + 1) * _G, _G)]],
                bufs[1 - b], sems[1 - b])
        descs[b].wait()
        pltpu.sync_copy(bufs[b], out_hbm.at[pl.ds(out_base + g * _G, _G)])


_sc_embed = functools.partial(
    pl.kernel,
    out_type=jax.ShapeDtypeStruct((_B * _S, _D), jnp.float32),
    mesh=plsc.VectorSubcoreMesh(core_axis_name="c", subcore_axis_name="s",
                                num_cores=_NC, num_subcores=_NS),
    scratch_types=[
        pltpu.VMEM((_S,), jnp.int32),
        pltpu.VMEM((_CH,), jnp.int32),
        pltpu.VMEM((_G, _D), jnp.float32),
        pltpu.VMEM((_G, _D), jnp.float32),
        pltpu.SemaphoreType.DMA,
        pltpu.SemaphoreType.DMA,
    ],
)(_sc_body)


def kernel(input_ids, weights):
    bsz, seq_len = input_ids.shape
    out = _sc_embed(input_ids, weights)
    return out.reshape(bsz, seq_len, -1)


# speedup vs baseline: 1.6336x; 1.0027x over previous
import functools

import jax
import jax.numpy as jnp
from jax import lax
from jax.experimental import pallas as pl
from jax.experimental.pallas import tpu as pltpu
from jax.experimental.pallas import tpu_sc as plsc

_PAD = 1

_B, _S, _D = 4, 4096, 1024
_NC, _NS = 2, 16
_NW = _NC * _NS
_CH = (_B * _S) // _NW
_ROWS_PER_CORE = _B // _NC
_CHUNKS_PER_ROW = _NS // _ROWS_PER_CORE
_G = 32
_NG = _CH // _G
_L = 16

_LANE = None


def _lane_cumsum(x):
    lane = lax.iota(jnp.int32, _L)
    cs = x
    for d in (1, 2, 4, 8):
        idx = jnp.maximum(lane - d, 0)
        sh = cs.at[idx].get(mode="promise_in_bounds")
        cs = cs + jnp.where(lane >= d, sh, 0)
    return cs


def _splat_last(x):
    return x.at[jnp.full((_L,), _L - 1, jnp.int32)].get(
        mode="promise_in_bounds")


def _sc_body(ids_hbm, w_hbm, out_hbm, ids_v, idx_v, rows0_v, rows1_v,
             rows2_v, gsem0, gsem1, gsem2, ssem0, ssem1, ssem2):
    c = lax.axis_index("c")
    s = lax.axis_index("s")
    row = c * _ROWS_PER_CORE + s // _CHUNKS_PER_ROW
    chunk = s % _CHUNKS_PER_ROW

    pltpu.sync_copy(ids_hbm.at[row], ids_v)

    def pref_body(j, acc):
        v = ids_v[pl.ds(j * _L, _L)]
        return acc + jnp.where(v != _PAD, jnp.int32(1), jnp.int32(0))

    acc = lax.fori_loop(0, chunk * (_CH // _L), pref_body,
                        jnp.zeros((_L,), jnp.int32))
    offset = _splat_last(_lane_cumsum(acc))

    chunk_base = chunk * _CH

    def pos_body(j, off):
        v = ids_v[pl.ds(chunk_base + j * _L, _L)]
        mi = jnp.where(v != _PAD, jnp.int32(1), jnp.int32(0))
        cs = _lane_cumsum(mi) + off
        idx_v[pl.ds(j * _L, _L)] = cs * mi + _PAD
        return _splat_last(cs)

    lax.fori_loop(0, _CH // _L, pos_body, offset)

    out_base = row * _S + chunk_base
    bufs = (rows0_v, rows1_v, rows2_v)
    gsems = (gsem0, gsem1, gsem2)
    ssems = (ssem0, ssem1, ssem2)

    def gather(g):
        return pltpu.async_copy(w_hbm.at[idx_v.at[pl.ds(g * _G, _G)]],
                                bufs[g % 3], gsems[g % 3])

    def scatter(g):
        return pltpu.async_copy(bufs[g % 3],
                                out_hbm.at[pl.ds(out_base + g * _G, _G)],
                                ssems[g % 3])

    gdescs = [None, None, None]
    sdescs = [None, None, None]
    gdescs[0] = gather(0)
    gdescs[1] = gather(1)
    for g in range(_NG):
        b = g % 3
        if g + 2 < _NG:
            if g >= 1:
                sdescs[(g - 1) % 3].wait()
            gdescs[(g + 2) % 3] = gather(g + 2)
        gdescs[b].wait()
        sdescs[b] = scatter(g)
    sdescs[(_NG - 3) % 3].wait()
    sdescs[(_NG - 2) % 3].wait()
    sdescs[(_NG - 1) % 3].wait()


_sc_embed = functools.partial(
    pl.kernel,
    out_type=jax.ShapeDtypeStruct((_B * _S, _D), jnp.float32),
    mesh=plsc.VectorSubcoreMesh(core_axis_name="c", subcore_axis_name="s",
                                num_cores=_NC, num_subcores=_NS),
    scratch_types=[
        pltpu.VMEM((_S,), jnp.int32),
        pltpu.VMEM((_CH,), jnp.int32),
        pltpu.VMEM((_G, _D), jnp.float32),
        pltpu.VMEM((_G, _D), jnp.float32),
        pltpu.VMEM((_G, _D), jnp.float32),
        pltpu.SemaphoreType.DMA,
        pltpu.SemaphoreType.DMA,
        pltpu.SemaphoreType.DMA,
        pltpu.SemaphoreType.DMA,
        pltpu.SemaphoreType.DMA,
        pltpu.SemaphoreType.DMA,
    ],
)(_sc_body)


def kernel(input_ids, weights):
    bsz, seq_len = input_ids.shape
    out = _sc_embed(input_ids, weights)
    return out.reshape(bsz, seq_len, -1)
